# speedup vs baseline: 2.9720x; 2.9720x over previous
import jax
import jax.numpy as jnp
from jax import lax
from jax.experimental import pallas as pl
from jax.experimental.pallas import tpu as pltpu
from jax.experimental.pallas import tpu_sc as plsc

NC = 2
NS = 16
NW = NC * NS
CH = 128


def _emb_body(table_hbm, idx_hbm, out_hbm, idx_v, buf, sem):
    wid = lax.axis_index("s") * NC + lax.axis_index("c")
    n_ch = idx_hbm.shape[1]
    pltpu.sync_copy(idx_hbm.at[wid], idx_v)
    base = wid * (n_ch * CH)

    @pl.loop(0, n_ch)
    def chunk(j):
        pltpu.async_copy(table_hbm.at[idx_v.at[j]], buf, sem).wait()
        pltpu.sync_copy(buf, out_hbm.at[pl.ds(base + j * CH, CH)])


def kernel(x, W):
    B, S = x.shape
    V, D = W.shape
    total = B * S
    n_ch = total // (NW * CH)
    idx = x.reshape(NW, n_ch, CH).astype(jnp.int32)
    mesh = plsc.VectorSubcoreMesh(core_axis_name="c", subcore_axis_name="s")
    run = pl.kernel(
        _emb_body,
        out_type=jax.ShapeDtypeStruct((total, D), jnp.float32),
        mesh=mesh,
        scratch_types=[
            pltpu.VMEM((n_ch, CH), jnp.int32),
            pltpu.VMEM((CH, D), jnp.float32),
            pltpu.SemaphoreType.DMA,
        ],
    )
    out = run(W, idx)
    return out.reshape(B, S, D)


# speedup vs baseline: 3.3541x; 1.1286x over previous
import jax
import jax.numpy as jnp
from jax import lax
from jax.experimental import pallas as pl
from jax.experimental.pallas import tpu as pltpu
from jax.experimental.pallas import tpu_sc as plsc

NC = 2
NS = 16
NW = NC * NS
CH = 128


NBUF = 4


def _emb_body(table_hbm, idx_hbm, out_hbm, idx_v, bufs, gsem, ssem):
    wid = lax.axis_index("s") * NC + lax.axis_index("c")
    n_ch = idx_hbm.shape[1]
    pltpu.sync_copy(idx_hbm.at[wid], idx_v)
    base = wid * (n_ch * CH)

    def start_gather(c, b):
        pltpu.make_async_copy(table_hbm.at[idx_v.at[c]], bufs.at[b], gsem).start()

    def wait_gather():
        pltpu.make_async_copy(out_hbm.at[pl.ds(0, CH)], bufs.at[0], gsem).wait()

    def wait_scatter():
        pltpu.make_async_copy(bufs.at[0], out_hbm.at[pl.ds(0, CH)], ssem).wait()

    for b in range(NBUF - 1):
        start_gather(b, b)

    @pl.loop(0, n_ch)
    def chunk(c):
        b = lax.rem(c, NBUF)
        wait_gather()
        pltpu.make_async_copy(
            bufs.at[b], out_hbm.at[pl.ds(base + c * CH, CH)], ssem).start()

        @pl.when(c >= 1)
        def _():
            wait_scatter()

        @pl.when(c + (NBUF - 1) < n_ch)
        def _():
            start_gather(c + (NBUF - 1), lax.rem(c + (NBUF - 1), NBUF))

    wait_scatter()


def kernel(x, W):
    B, S = x.shape
    V, D = W.shape
    total = B * S
    n_ch = total // (NW * CH)
    idx = x.reshape(NW, n_ch, CH).astype(jnp.int32)
    mesh = plsc.VectorSubcoreMesh(core_axis_name="c", subcore_axis_name="s")
    run = pl.kernel(
        _emb_body,
        out_type=jax.ShapeDtypeStruct((total, D), jnp.float32),
        mesh=mesh,
        scratch_types=[
            pltpu.VMEM((n_ch, CH), jnp.int32),
            pltpu.VMEM((NBUF, CH, D), jnp.float32),
            pltpu.SemaphoreType.DMA,
            pltpu.SemaphoreType.DMA,
        ],
    )
    out = run(W, idx)
    return out.reshape(B, S, D)
